# Initial kernel scaffold; baseline (speedup 1.0000x reference)
import functools

import jax
import jax.numpy as jnp
from jax import lax
from jax.experimental import pallas as pl
from jax.experimental.pallas import tpu as pltpu
from jax.experimental.pallas import tpu_sc as plsc

N = 10000
E = 320000
D = 128
H = 64
G = 200

NC = 2
NS = 16
NW = NC * NS
CHUNK = 128
CPT = 79
E_PAD = NW * CPT * CHUNK
N_PAD = 10240
ROWS_PER_TILE = N_PAD // NS
ZSTEPS = ROWS_PER_TILE // CHUNK


def _make_sc_agg(d):
    mesh = plsc.VectorSubcoreMesh(core_axis_name="c", subcore_axis_name="s")

    @functools.partial(
        pl.kernel,
        out_type=jax.ShapeDtypeStruct((NC * N_PAD, d), jnp.float32),
        mesh=mesh,
        scratch_types=[
            pltpu.VMEM((CPT, CHUNK), jnp.int32),
            pltpu.VMEM((CPT, CHUNK), jnp.int32),
            pltpu.VMEM((CHUNK, d), jnp.float32),
            pltpu.VMEM_SHARED((N_PAD, d), jnp.float32),
        ],
    )
    def sc_agg(h_hbm, src_hbm, dst_hbm, out_hbm, src_v, dst_v, rows_v, acc_sh):
        cid = lax.axis_index("c")
        sid = lax.axis_index("s")
        wid = cid * NS + sid

        @pl.loop(0, CHUNK)
        def _(r):
            @pl.loop(0, d, step=16)
            def _(c):
                rows_v[r, pl.ds(c, 16)] = jnp.zeros((16,), jnp.float32)

        @pl.loop(0, ZSTEPS)
        def _(k):
            pltpu.sync_copy(
                rows_v,
                acc_sh.at[pl.ds(sid * ROWS_PER_TILE + k * CHUNK, CHUNK)],
            )

        plsc.subcore_barrier()

        pltpu.sync_copy(src_hbm.at[pl.ds(wid * CPT, CPT)], src_v)
        pltpu.sync_copy(dst_hbm.at[pl.ds(wid * CPT, CPT)], dst_v)

        @pl.loop(0, CPT)
        def _(j):
            pltpu.sync_copy(h_hbm.at[src_v.at[j]], rows_v)
            pltpu.sync_copy(rows_v, acc_sh.at[dst_v.at[j]], add=True)

        plsc.subcore_barrier()

        pltpu.sync_copy(
            acc_sh.at[pl.ds(sid * ROWS_PER_TILE, ROWS_PER_TILE)],
            out_hbm.at[pl.ds(cid * N_PAD + sid * ROWS_PER_TILE, ROWS_PER_TILE)],
        )

    return sc_agg


_sc_agg_d = _make_sc_agg(D)
_sc_agg_h = _make_sc_agg(H)


def _dense_part(h, parts, eps, wa, ba, wb, bb, g, bt):
    agg = parts[0:N, :] + parts[N_PAD:N_PAD + N, :]
    z = (1.0 + eps[0, 0]) * h + agg
    t = jnp.dot(z, wa, preferred_element_type=jnp.float32,
                precision=lax.Precision.HIGHEST) + ba
    t = jnp.maximum(t, 0.0)
    t = jnp.dot(t, wb, preferred_element_type=jnp.float32,
                precision=lax.Precision.HIGHEST) + bb
    t = jnp.maximum(t, 0.0)
    m = jnp.mean(t, axis=0, keepdims=True)
    v = jnp.mean(t * t, axis=0, keepdims=True) - m * m
    return (t - m) * lax.rsqrt(v + 1e-5) * g + bt


def _tc_layer_body(h_ref, parts_ref, eps_ref, wa_ref, ba_ref, wb_ref, bb_ref,
                   g_ref, bt_ref, o_ref):
    o_ref[...] = _dense_part(
        h_ref[...], parts_ref[...], eps_ref[...], wa_ref[...], ba_ref[...],
        wb_ref[...], bb_ref[...], g_ref[...], bt_ref[...])


def _tc_final_body(h_ref, parts_ref, eps_ref, wa_ref, ba_ref, wb_ref, bb_ref,
                   g_ref, bt_ref, batch_ref, f1w_ref, f1b_ref, f2w_ref,
                   f2b_ref, o_ref):
    t = _dense_part(
        h_ref[...], parts_ref[...], eps_ref[...], wa_ref[...], ba_ref[...],
        wb_ref[...], bb_ref[...], g_ref[...], bt_ref[...])
    onehot = (batch_ref[...] ==
              lax.broadcasted_iota(jnp.int32, (N, G), 1)).astype(jnp.float32)
    pooled = lax.dot_general(onehot, t, (((0,), (0,)), ((), ())),
                             preferred_element_type=jnp.float32,
                             precision=lax.Precision.HIGHEST)
    p = jnp.maximum(jnp.dot(pooled, f1w_ref[...],
                            preferred_element_type=jnp.float32,
                            precision=lax.Precision.HIGHEST) + f1b_ref[...],
                    0.0)
    o_ref[...] = jnp.sum(p * f2w_ref[...], axis=1, keepdims=True) + f2b_ref[...]


def _tc_layer(h, parts, p):
    wa, ba, wb, bb, eps, g, bt = p
    return pl.pallas_call(
        _tc_layer_body,
        out_shape=jax.ShapeDtypeStruct((N, H), jnp.float32),
    )(h, parts, eps.reshape(1, 1), wa, ba.reshape(1, H), wb, bb.reshape(1, H),
      g.reshape(1, H), bt.reshape(1, H))


def _tc_final(h, parts, p, batch, fc1, fc2):
    wa, ba, wb, bb, eps, g, bt = p
    return pl.pallas_call(
        _tc_final_body,
        out_shape=jax.ShapeDtypeStruct((G, 1), jnp.float32),
    )(h, parts, eps.reshape(1, 1), wa, ba.reshape(1, H), wb, bb.reshape(1, H),
      g.reshape(1, H), bt.reshape(1, H), batch.reshape(N, 1),
      fc1[0], fc1[1].reshape(1, 32), fc2[0].reshape(1, 32),
      fc2[1].reshape(1, 1))


def kernel(x, edge_index, batch, p1, p2, p3, p4, fc1, fc2):
    src = edge_index[0]
    dst = edge_index[1]
    pad = E_PAD - E
    src_p = jnp.concatenate(
        [src, jnp.zeros((pad,), jnp.int32)]).reshape(E_PAD // CHUNK, CHUNK)
    dst_p = jnp.concatenate(
        [dst, jnp.full((pad,), N, jnp.int32)]).reshape(E_PAD // CHUNK, CHUNK)

    h = x
    parts = _sc_agg_d(h, src_p, dst_p)
    h = _tc_layer(h, parts, p1)
    for p in (p2, p3):
        parts = _sc_agg_h(h, src_p, dst_p)
        h = _tc_layer(h, parts, p)
    parts = _sc_agg_h(h, src_p, dst_p)
    return _tc_final(h, parts, p4, batch, fc1, fc2)


# speedup vs baseline: 2.5375x; 2.5375x over previous
import functools

import jax
import jax.numpy as jnp
from jax import lax
from jax.experimental import pallas as pl
from jax.experimental.pallas import tpu as pltpu
from jax.experimental.pallas import tpu_sc as plsc

N = 10000
E = 320000
D = 128
H = 64
G = 200

NC = 2
NS = 16
NW = NC * NS
CHUNK = 128
CPT = 80
E_PAD = NW * CPT * CHUNK
N_PAD = 10240
ROWS_PER_TILE = N_PAD // NS
ZSTEPS = ROWS_PER_TILE // CHUNK

_MESH = plsc.VectorSubcoreMesh(core_axis_name="c", subcore_axis_name="s")


@functools.partial(
    pl.kernel,
    out_type=jax.ShapeDtypeStruct((NC * N_PAD, D), jnp.float32),
    mesh=_MESH,
    scratch_types=[
        pltpu.VMEM((CPT, CHUNK), jnp.int32),
        pltpu.VMEM((CPT, CHUNK), jnp.int32),
        pltpu.VMEM((CHUNK, D), jnp.float32),
        pltpu.VMEM_SHARED((N_PAD, D), jnp.float32),
    ],
)
def _sc_agg(h_hbm, src_hbm, dst_hbm, out_hbm, src_v, dst_v, rows_v, acc_sh):
    cid = lax.axis_index("c")
    sid = lax.axis_index("s")
    wid = cid * NS + sid

    @pl.loop(0, CHUNK)
    def _(r):
        @pl.loop(0, D, step=16)
        def _(c):
            rows_v[r, pl.ds(c, 16)] = jnp.zeros((16,), jnp.float32)

    @pl.loop(0, ZSTEPS)
    def _(k):
        pltpu.sync_copy(
            rows_v,
            acc_sh.at[pl.ds(sid * ROWS_PER_TILE + k * CHUNK, CHUNK)],
        )

    plsc.subcore_barrier()

    pltpu.sync_copy(src_hbm.at[pl.ds(wid * CPT, CPT)], src_v)
    pltpu.sync_copy(dst_hbm.at[pl.ds(wid * CPT, CPT)], dst_v)

    @pl.loop(0, CPT)
    def _(j):
        pltpu.sync_copy(h_hbm.at[src_v.at[j]], rows_v)
        pltpu.sync_copy(rows_v, acc_sh.at[dst_v.at[j]], add=True)

    plsc.subcore_barrier()

    pltpu.sync_copy(
        acc_sh.at[pl.ds(sid * ROWS_PER_TILE, ROWS_PER_TILE)],
        out_hbm.at[pl.ds(cid * N_PAD + sid * ROWS_PER_TILE, ROWS_PER_TILE)],
    )


def _bf16_dot(a, b):
    return jnp.dot(a.astype(jnp.bfloat16), b.astype(jnp.bfloat16),
                   preferred_element_type=jnp.float32)


def _dense_part(h, parts, eps, wa, ba, wb, bb, g, bt, din):
    agg = parts[0:N, 0:din] + parts[N_PAD:N_PAD + N, 0:din]
    z = (1.0 + eps[0, 0]) * h[:, 0:din] + agg
    t = _bf16_dot(z, wa) + ba
    t = jnp.maximum(t, 0.0)
    t = _bf16_dot(t, wb) + bb
    t = jnp.maximum(t, 0.0)
    m = jnp.mean(t, axis=0, keepdims=True)
    c = t - m
    v = jnp.mean(c * c, axis=0, keepdims=True)
    return c / jnp.sqrt(v + 1e-5) * g + bt


def _tc_layer_body(din, h_ref, parts_ref, eps_ref, wa_ref, ba_ref, wb_ref,
                   bb_ref, g_ref, bt_ref, o_ref):
    t = _dense_part(
        h_ref[...], parts_ref[...], eps_ref[...], wa_ref[...], ba_ref[...],
        wb_ref[...], bb_ref[...], g_ref[...], bt_ref[...], din)
    o_ref[...] = jnp.concatenate(
        [t, jnp.zeros((N, D - H), jnp.float32)], axis=1)


def _tc_final_body(h_ref, parts_ref, eps_ref, wa_ref, ba_ref, wb_ref, bb_ref,
                   g_ref, bt_ref, batch_ref, f1w_ref, f1b_ref, f2w_ref,
                   f2b_ref, o_ref):
    t = _dense_part(
        h_ref[...], parts_ref[...], eps_ref[...], wa_ref[...], ba_ref[...],
        wb_ref[...], bb_ref[...], g_ref[...], bt_ref[...], H)
    onehot = (batch_ref[...] ==
              lax.broadcasted_iota(jnp.int32, (N, G), 1)).astype(jnp.float32)
    pooled = lax.dot_general(onehot, t, (((0,), (0,)), ((), ())),
                             preferred_element_type=jnp.float32,
                             precision=lax.Precision.HIGHEST)
    p = jnp.maximum(_bf16_dot(pooled, f1w_ref[...]) + f1b_ref[...], 0.0)
    prod = (p.astype(jnp.bfloat16) *
            f2w_ref[...].astype(jnp.bfloat16)).astype(jnp.float32)
    o_ref[...] = jnp.sum(prod, axis=1, keepdims=True) + f2b_ref[...]


def _tc_layer(h, parts, p, din):
    wa, ba, wb, bb, eps, g, bt = p
    return pl.pallas_call(
        functools.partial(_tc_layer_body, din),
        out_shape=jax.ShapeDtypeStruct((N, D), jnp.float32),
    )(h, parts, eps.reshape(1, 1), wa, ba.reshape(1, H), wb, bb.reshape(1, H),
      g.reshape(1, H), bt.reshape(1, H))


def _tc_final(h, parts, p, batch, fc1, fc2):
    wa, ba, wb, bb, eps, g, bt = p
    return pl.pallas_call(
        _tc_final_body,
        out_shape=jax.ShapeDtypeStruct((G, 1), jnp.float32),
    )(h, parts, eps.reshape(1, 1), wa, ba.reshape(1, H), wb, bb.reshape(1, H),
      g.reshape(1, H), bt.reshape(1, H), batch.reshape(N, 1),
      fc1[0], fc1[1].reshape(1, 32), fc2[0].reshape(1, 32),
      fc2[1].reshape(1, 1))


def kernel(x, edge_index, batch, p1, p2, p3, p4, fc1, fc2):
    src = edge_index[0]
    dst = edge_index[1]
    pad = E_PAD - E
    src_p = jnp.concatenate(
        [src, jnp.zeros((pad,), jnp.int32)]).reshape(E_PAD // CHUNK, CHUNK)
    dst_p = jnp.concatenate(
        [dst, jnp.full((pad,), N, jnp.int32)]).reshape(E_PAD // CHUNK, CHUNK)

    parts = _sc_agg(x, src_p, dst_p)
    h = _tc_layer(x, parts, p1, D)
    for p in (p2, p3):
        parts = _sc_agg(h, src_p, dst_p)
        h = _tc_layer(h, parts, p, H)
    parts = _sc_agg(h, src_p, dst_p)
    return _tc_final(h, parts, p4, batch, fc1, fc2)


# speedup vs baseline: 2.5480x; 1.0041x over previous
import functools

import jax
import jax.numpy as jnp
from jax import lax
from jax.experimental import pallas as pl
from jax.experimental.pallas import tpu as pltpu
from jax.experimental.pallas import tpu_sc as plsc

N = 10000
E = 320000
D = 128
H = 64
G = 200

NC = 2
NS = 16
NW = NC * NS
CHUNK = 128
CPT = 80
E_PAD = NW * CPT * CHUNK
N_PAD = 10240
ROWS_PER_TILE = N_PAD // NS
ZSTEPS = ROWS_PER_TILE // CHUNK

_MESH = plsc.VectorSubcoreMesh(core_axis_name="c", subcore_axis_name="s")


@functools.partial(
    pl.kernel,
    out_type=jax.ShapeDtypeStruct((NC * N_PAD, D), jnp.float32),
    mesh=_MESH,
    scratch_types=[
        pltpu.VMEM((CPT, CHUNK), jnp.int32),
        pltpu.VMEM((CPT, CHUNK), jnp.int32),
        pltpu.VMEM((CHUNK, D), jnp.float32),
        pltpu.VMEM_SHARED((N_PAD, D), jnp.float32),
    ],
)
def _sc_agg(h_hbm, src_hbm, dst_hbm, out_hbm, src_v, dst_v, rows_v, acc_sh):
    cid = lax.axis_index("c")
    sid = lax.axis_index("s")
    wid = cid * NS + sid

    @pl.loop(0, CHUNK)
    def _(r):
        @pl.loop(0, D, step=16)
        def _(c):
            rows_v[r, pl.ds(c, 16)] = jnp.zeros((16,), jnp.float32)

    @pl.loop(0, ZSTEPS)
    def _(k):
        pltpu.sync_copy(
            rows_v,
            acc_sh.at[pl.ds(sid * ROWS_PER_TILE + k * CHUNK, CHUNK)],
        )

    plsc.subcore_barrier()

    pltpu.sync_copy(src_hbm.at[pl.ds(wid * CPT, CPT)], src_v)
    pltpu.sync_copy(dst_hbm.at[pl.ds(wid * CPT, CPT)], dst_v)

    @pl.loop(0, CPT)
    def _(j):
        pltpu.sync_copy(h_hbm.at[src_v.at[j]], rows_v)
        pltpu.sync_copy(rows_v, acc_sh.at[dst_v.at[j]], add=True)

    plsc.subcore_barrier()

    pltpu.sync_copy(
        acc_sh.at[pl.ds(sid * ROWS_PER_TILE, ROWS_PER_TILE)],
        out_hbm.at[pl.ds(cid * N_PAD + sid * ROWS_PER_TILE, ROWS_PER_TILE)],
    )


def _bf16_dot(a, b):
    return jnp.dot(a.astype(jnp.bfloat16), b.astype(jnp.bfloat16),
                   preferred_element_type=jnp.float32)


def _dense_part(h, parts, eps, wa, ba, wb, bb, g, bt, din):
    agg = parts[0:N, 0:din] + parts[N_PAD:N_PAD + N, 0:din]
    z = (1.0 + eps[0, 0]) * h[:, 0:din] + agg
    t = _bf16_dot(z, wa) + ba
    t = jnp.maximum(t, 0.0)
    t = _bf16_dot(t, wb) + bb
    t = jnp.maximum(t, 0.0)
    m = jnp.mean(t, axis=0, keepdims=True)
    v = jnp.mean(t * t, axis=0, keepdims=True) - m * m
    return (t - m) * lax.rsqrt(v + 1e-5) * g + bt


def _tc_layer_body(din, h_ref, parts_ref, eps_ref, wa_ref, ba_ref, wb_ref,
                   bb_ref, g_ref, bt_ref, o_ref):
    t = _dense_part(
        h_ref[...], parts_ref[...], eps_ref[...], wa_ref[...], ba_ref[...],
        wb_ref[...], bb_ref[...], g_ref[...], bt_ref[...], din)
    o_ref[...] = jnp.concatenate(
        [t, jnp.zeros((N, D - H), jnp.float32)], axis=1)


def _tc_final_body(h_ref, parts_ref, eps_ref, wa_ref, ba_ref, wb_ref, bb_ref,
                   g_ref, bt_ref, batch_ref, f1w_ref, f1b_ref, f2w_ref,
                   f2b_ref, o_ref):
    t = _dense_part(
        h_ref[...], parts_ref[...], eps_ref[...], wa_ref[...], ba_ref[...],
        wb_ref[...], bb_ref[...], g_ref[...], bt_ref[...], H)
    onehot = (batch_ref[...] ==
              lax.broadcasted_iota(jnp.int32, (N, G), 1)).astype(jnp.float32)
    pooled = lax.dot_general(onehot, t, (((0,), (0,)), ((), ())),
                             preferred_element_type=jnp.float32,
                             precision=lax.Precision.HIGHEST)
    p = jnp.maximum(_bf16_dot(pooled, f1w_ref[...]) + f1b_ref[...], 0.0)
    prod = (p.astype(jnp.bfloat16) *
            f2w_ref[...].astype(jnp.bfloat16)).astype(jnp.float32)
    o_ref[...] = jnp.sum(prod, axis=1, keepdims=True) + f2b_ref[...]


def _tc_layer(h, parts, p, din):
    wa, ba, wb, bb, eps, g, bt = p
    return pl.pallas_call(
        functools.partial(_tc_layer_body, din),
        out_shape=jax.ShapeDtypeStruct((N, D), jnp.float32),
    )(h, parts, eps.reshape(1, 1), wa, ba.reshape(1, H), wb, bb.reshape(1, H),
      g.reshape(1, H), bt.reshape(1, H))


def _tc_final(h, parts, p, batch, fc1, fc2):
    wa, ba, wb, bb, eps, g, bt = p
    return pl.pallas_call(
        _tc_final_body,
        out_shape=jax.ShapeDtypeStruct((G, 1), jnp.float32),
    )(h, parts, eps.reshape(1, 1), wa, ba.reshape(1, H), wb, bb.reshape(1, H),
      g.reshape(1, H), bt.reshape(1, H), batch.reshape(N, 1),
      fc1[0], fc1[1].reshape(1, 32), fc2[0].reshape(1, 32),
      fc2[1].reshape(1, 1))


def kernel(x, edge_index, batch, p1, p2, p3, p4, fc1, fc2):
    src = edge_index[0]
    dst = edge_index[1]
    pad = E_PAD - E
    src_p = jnp.concatenate(
        [src, jnp.zeros((pad,), jnp.int32)]).reshape(E_PAD // CHUNK, CHUNK)
    dst_p = jnp.concatenate(
        [dst, jnp.full((pad,), N, jnp.int32)]).reshape(E_PAD // CHUNK, CHUNK)

    parts = _sc_agg(x, src_p, dst_p)
    h = _tc_layer(x, parts, p1, D)
    for p in (p2, p3):
        parts = _sc_agg(h, src_p, dst_p)
        h = _tc_layer(h, parts, p, H)
    parts = _sc_agg(h, src_p, dst_p)
    return _tc_final(h, parts, p4, batch, fc1, fc2)
